# baseline (speedup 1.0000x reference)
import math

import jax
import jax.numpy as jnp
from jax import lax
from jax.experimental import pallas as pl
from jax.experimental.pallas import tpu as pltpu
from jax.experimental.pallas import tpu_sc as plsc

N = 10000
E = 160000
D = 256
H = 8
HD = 32
QW = 64
NQ = 4
NC = 2
NS = 16
NT = 10016
TRASH = 10000
ROWS_PER_TILE = NT // NS
CW = 72
C = 128
TILES = NC * NS
CPT = 40
EPAD = (TILES * CPT + 1) * C
INV_SQRT_HD = 1.0 / math.sqrt(HD)

NBLK = 1000
EBLK = 1000


def _proj_nodes_body(x_ref, w1_ref, w2_ref, w3_ref, w4_ref,
                     b1_ref, b2_ref, b3_ref, b4_ref,
                     o1_ref, *oq_refs):
    xb = x_ref[...]
    o1_ref[...] = (jnp.dot(xb, w1_ref[...], preferred_element_type=jnp.float32)
                   + b1_ref[...])
    h3 = jnp.dot(xb, w3_ref[...], preferred_element_type=jnp.float32) + b3_ref[...]
    h4 = jnp.dot(xb, w4_ref[...], preferred_element_type=jnp.float32) + b4_ref[...]
    h2 = jnp.dot(xb, w2_ref[...], preferred_element_type=jnp.float32) + b2_ref[...]
    for qi in range(NQ):
        sl = slice(QW * qi, QW * (qi + 1))
        oq_refs[qi][...] = h3[:, sl]
        oq_refs[NQ + qi][...] = jnp.concatenate([h4[:, sl], h2[:, sl]], axis=1)


def _proj_nodes(x, W1, b1, W2, b2, W3, b3, W4, b4):
    grid = (N // NBLK,)
    wspec = pl.BlockSpec((D, D), lambda i: (0, 0))
    bspec = pl.BlockSpec((1, D), lambda i: (0, 0))
    qspec = pl.BlockSpec((NBLK, QW), lambda i: (i, 0))
    kvspec = pl.BlockSpec((NBLK, 2 * QW), lambda i: (i, 0))
    return pl.pallas_call(
        _proj_nodes_body,
        grid=grid,
        in_specs=[pl.BlockSpec((NBLK, D), lambda i: (i, 0)),
                  wspec, wspec, wspec, wspec,
                  bspec, bspec, bspec, bspec],
        out_specs=[pl.BlockSpec((NBLK, D), lambda i: (i, 0))]
        + [qspec] * NQ + [kvspec] * NQ,
        out_shape=[jax.ShapeDtypeStruct((N, D), jnp.float32)]
        + [jax.ShapeDtypeStruct((N, QW), jnp.float32)] * NQ
        + [jax.ShapeDtypeStruct((N, 2 * QW), jnp.float32)] * NQ,
    )(x, W1, W2, W3, W4, b1, b2, b3, b4)


def _proj_edges_body(ea_ref, w5_ref, b5_ref, *o_refs):
    h5 = (jnp.dot(ea_ref[...], w5_ref[...],
                  preferred_element_type=jnp.float32) + b5_ref[...])
    for qi in range(NQ):
        o_refs[qi][...] = h5[:, QW * qi:QW * (qi + 1)]


def _proj_edges(edge_attr, W5, b5):
    grid = (E // EBLK,)
    return pl.pallas_call(
        _proj_edges_body,
        grid=grid,
        in_specs=[pl.BlockSpec((EBLK, D), lambda i: (i, 0)),
                  pl.BlockSpec((D, D), lambda i: (0, 0)),
                  pl.BlockSpec((1, D), lambda i: (0, 0))],
        out_specs=[pl.BlockSpec((EBLK, QW), lambda i: (i, 0))] * NQ,
        out_shape=[jax.ShapeDtypeStruct((E, QW), jnp.float32)] * NQ,
    )(edge_attr, W5, b5)


def _sc_body(dst_hbm, src_hbm,
             q0, kv0, w50, q1, kv1, w51, q2, kv2, w52, q3, kv3, w53,
             out_hbm,
             idxd0, idxd1, idxs0, idxs1,
             qv0, qv1, kvv0, kvv1, wv0, wv1, uv0, uv1, acc_sh,
             semg0, semg1, semi):
    idxd = [idxd0, idxd1]
    idxs = [idxs0, idxs1]
    qv = [qv0, qv1]
    kvv = [kvv0, kvv1]
    wv = [wv0, wv1]
    uv = [uv0, uv1]
    semg = [semg0, semg1]

    cid = lax.axis_index("c")
    sid = lax.axis_index("s")
    zeros16 = jnp.zeros((16,), jnp.float32)
    iota16 = lax.iota(jnp.int32, 16)

    base_row = sid * ROWS_PER_TILE
    t = cid * NS + sid
    start = t * CPT

    def run_pass(qi, q_hbm, kv_hbm, w5_hbm):
        for sl in range(2):
            uu = uv[sl]

            @pl.loop(0, C)
            def _z(r, uu=uu):
                for j in range(CW // 16):
                    uu[r, pl.ds(j * 16, 16)] = zeros16
                uu[r, pl.ds(CW - 16, 16)] = zeros16

        for j in range(ROWS_PER_TILE // C):
            pltpu.sync_copy(uv[0], acc_sh.at[pl.ds(base_row + j * C, C)])
        rem = ROWS_PER_TILE % C
        if rem:
            pltpu.sync_copy(
                uv[0].at[pl.ds(0, rem)],
                acc_sh.at[pl.ds(base_row + (ROWS_PER_TILE // C) * C, rem)])
        plsc.subcore_barrier()

        def issue_gathers(sl, base):
            wbase = jnp.minimum(base, E - C)
            pltpu.async_copy(q_hbm.at[idxd[sl]], qv[sl], semg[sl])
            pltpu.async_copy(kv_hbm.at[idxs[sl]], kvv[sl], semg[sl])
            pltpu.async_copy(w5_hbm.at[pl.ds(wbase, C)], wv[sl], semg[sl])

        def wait_gathers(sl, base):
            wbase = jnp.minimum(base, E - C)
            pltpu.make_async_copy(q_hbm.at[idxd[sl]], qv[sl], semg[sl]).wait()
            pltpu.make_async_copy(kv_hbm.at[idxs[sl]], kvv[sl], semg[sl]).wait()
            pltpu.make_async_copy(w5_hbm.at[pl.ds(wbase, C)], wv[sl],
                                  semg[sl]).wait()

        def compute(sl):
            uu = uv[sl]
            qq = qv[sl]
            kk = kvv[sl]
            ww = wv[sl]

            @pl.loop(0, C, unroll=2)
            def _edge(r):
                qr = [qq[r, pl.ds(16 * i, 16)] for i in range(4)]
                kr = [kk[r, pl.ds(16 * i, 16)] for i in range(4)]
                vr = [kk[r, pl.ds(QW + 16 * i, 16)] for i in range(4)]
                wr = [ww[r, pl.ds(16 * i, 16)] for i in range(4)]
                ps = []
                for h in range(2):
                    th = (qr[2 * h] * (kr[2 * h] + wr[2 * h])
                          + qr[2 * h + 1] * (kr[2 * h + 1] + wr[2 * h + 1]))
                    s = jnp.sum(th) * INV_SQRT_HD
                    ps.append(jnp.exp(zeros16 + s))
                for i in range(4):
                    uu[r, pl.ds(16 * i, 16)] = (vr[i] + wr[i]) * ps[i // 2]
                pd = (jnp.where(iota16 == 0, ps[0], 0.0)
                      + jnp.where(iota16 == 1, ps[1], 0.0))
                plsc.store_scatter(uu, [iota16 * 0 + r, QW + (iota16 & 1)],
                                   pd, mask=iota16 < 2)

        pltpu.sync_copy(dst_hbm.at[pl.ds(start * C, C)], idxd[0])
        pltpu.sync_copy(src_hbm.at[pl.ds(start * C, C)], idxs[0])
        issue_gathers(0, start * C)

        @pl.loop(0, CPT // 2)
        def _pair(pi):
            for b in range(2):
                sl = b
                nx = 1 - b
                ci = 2 * pi + b
                base = (start + ci) * C
                nbase = base + C

                pltpu.async_copy(dst_hbm.at[pl.ds(nbase, C)], idxd[nx], semi)
                pltpu.async_copy(src_hbm.at[pl.ds(nbase, C)], idxs[nx], semi)

                wait_gathers(sl, base)

                compute(sl)

                pltpu.make_async_copy(dst_hbm.at[pl.ds(nbase, C)], idxd[nx],
                                      semi).wait()
                pltpu.make_async_copy(src_hbm.at[pl.ds(nbase, C)], idxs[nx],
                                      semi).wait()
                issue_gathers(nx, nbase)

                pltpu.sync_copy(uv[sl], acc_sh.at[idxd[sl]], add=True)

        wait_gathers(0, (start + CPT) * C)

        plsc.subcore_barrier()
        pltpu.sync_copy(acc_sh.at[pl.ds(base_row, ROWS_PER_TILE)],
                        out_hbm.at[qi, cid, pl.ds(base_row, ROWS_PER_TILE)])

    run_pass(0, q0, kv0, w50)
    run_pass(1, q1, kv1, w51)
    run_pass(2, q2, kv2, w52)
    run_pass(3, q3, kv3, w53)


def _sc_edges(dst, src, qs, kvs, w5s):
    mesh = plsc.VectorSubcoreMesh(core_axis_name="c", subcore_axis_name="s",
                                  num_cores=NC, num_subcores=NS)
    f = pl.kernel(
        _sc_body,
        out_type=jax.ShapeDtypeStruct((NQ, NC, NT, CW), jnp.float32),
        mesh=mesh,
        compiler_params=pltpu.CompilerParams(use_tc_tiling_on_sc=False,
                                             needs_layout_passes=False),
        scratch_types=[
            pltpu.VMEM((C,), jnp.int32),
            pltpu.VMEM((C,), jnp.int32),
            pltpu.VMEM((C,), jnp.int32),
            pltpu.VMEM((C,), jnp.int32),
            pltpu.VMEM((C, QW), jnp.float32),
            pltpu.VMEM((C, QW), jnp.float32),
            pltpu.VMEM((C, 2 * QW), jnp.float32),
            pltpu.VMEM((C, 2 * QW), jnp.float32),
            pltpu.VMEM((C, QW), jnp.float32),
            pltpu.VMEM((C, QW), jnp.float32),
            pltpu.VMEM((C, CW), jnp.float32),
            pltpu.VMEM((C, CW), jnp.float32),
            pltpu.VMEM_SHARED((NT, CW), jnp.float32),
            pltpu.SemaphoreType.DMA,
            pltpu.SemaphoreType.DMA,
            pltpu.SemaphoreType.DMA,
        ],
    )
    return f(dst, src,
             qs[0], kvs[0], w5s[0], qs[1], kvs[1], w5s[1],
             qs[2], kvs[2], w5s[2], qs[3], kvs[3], w5s[3])


def _combine_body(w1_ref, p0_ref, p1_ref, p2_ref, p3_ref, out_ref):
    ri = lax.broadcasted_iota(jnp.int32, (2, QW), 0)
    ci = lax.broadcasted_iota(jnp.int32, (2, QW), 1)
    expand = (ci // HD == ri).astype(jnp.float32)
    outs = []
    for p_ref in (p0_ref, p1_ref, p2_ref, p3_ref):
        p = p_ref[0, 0] + p_ref[0, 1]
        agg = p[:, :QW]
        den = p[:, QW:QW + 2]
        recip = jnp.where(den != 0.0, 1.0 / den, 0.0)
        outs.append(agg * jnp.dot(recip, expand,
                                  preferred_element_type=jnp.float32))
    out_ref[...] = w1_ref[...] + jnp.concatenate(outs, axis=1)


def _combine(w1f, parts):
    grid = (N // NBLK,)

    def pspec(qi):
        return pl.BlockSpec((1, NC, NBLK, CW), lambda i, qi=qi: (qi, 0, i, 0))

    return pl.pallas_call(
        _combine_body,
        grid=grid,
        in_specs=[pl.BlockSpec((NBLK, D), lambda i: (i, 0)),
                  pspec(0), pspec(1), pspec(2), pspec(3)],
        out_specs=pl.BlockSpec((NBLK, D), lambda i: (i, 0)),
        out_shape=jax.ShapeDtypeStruct((N, D), jnp.float32),
    )(w1f, parts, parts, parts, parts)


def kernel(x, edge_index, edge_attr, W1, b1, W2, b2, W3, b3, W4, b4, W5, b5):
    b1r = b1.reshape(1, D)
    b2r = b2.reshape(1, D)
    b3r = b3.reshape(1, D)
    b4r = b4.reshape(1, D)
    b5r = b5.reshape(1, D)
    dst = jnp.concatenate(
        [edge_index[0], jnp.full((EPAD - E,), TRASH, dtype=jnp.int32)])
    src = jnp.concatenate(
        [edge_index[1], jnp.zeros((EPAD - E,), dtype=jnp.int32)])

    w1f, *nodes = _proj_nodes(x, W1, b1r, W2, b2r, W3, b3r, W4, b4r)
    zq = jnp.zeros((NT - N, QW), jnp.float32)
    qs = [jnp.concatenate([q, zq]) for q in nodes[:NQ]]
    kvs = nodes[NQ:]
    w5s = _proj_edges(edge_attr, W5, b5r)

    parts = _sc_edges(dst, src, qs, kvs, w5s)

    return _combine(w1f, parts)


# speedup vs baseline: 1.5220x; 1.5220x over previous
import math

import jax
import jax.numpy as jnp
from jax import lax
from jax.experimental import pallas as pl
from jax.experimental.pallas import tpu as pltpu
from jax.experimental.pallas import tpu_sc as plsc

N = 10000
E = 160000
D = 256
H = 8
HD = 32
QW = 64
NQ = 4
NC = 2
NS = 16
NT = 10016
TRASH = 10000
ROWS_PER_TILE = NT // NS
CW = 72
C = 128
TILES = NC * NS
CPT = 40
EPAD = (TILES * CPT + 1) * C
INV_SQRT_HD = 1.0 / math.sqrt(HD)

NBLK = 1000
EBLK = 1000


def _proj_nodes_body(x_ref, w1_ref, w2_ref, w3_ref, w4_ref,
                     b1_ref, b2_ref, b3_ref, b4_ref,
                     o1_ref, *oq_refs):
    xb = x_ref[...]
    o1_ref[...] = (jnp.dot(xb, w1_ref[...], preferred_element_type=jnp.float32)
                   + b1_ref[...])
    h3 = jnp.dot(xb, w3_ref[...], preferred_element_type=jnp.float32) + b3_ref[...]
    h4 = jnp.dot(xb, w4_ref[...], preferred_element_type=jnp.float32) + b4_ref[...]
    h2 = jnp.dot(xb, w2_ref[...], preferred_element_type=jnp.float32) + b2_ref[...]
    for qi in range(NQ):
        sl = slice(QW * qi, QW * (qi + 1))
        oq_refs[qi][...] = h3[:, sl]
        oq_refs[NQ + qi][...] = jnp.concatenate([h4[:, sl], h2[:, sl]], axis=1)


def _proj_nodes(x, W1, b1, W2, b2, W3, b3, W4, b4):
    grid = (N // NBLK,)
    wspec = pl.BlockSpec((D, D), lambda i: (0, 0))
    bspec = pl.BlockSpec((1, D), lambda i: (0, 0))
    qspec = pl.BlockSpec((NBLK, QW), lambda i: (i, 0))
    kvspec = pl.BlockSpec((NBLK, 2 * QW), lambda i: (i, 0))
    return pl.pallas_call(
        _proj_nodes_body,
        grid=grid,
        in_specs=[pl.BlockSpec((NBLK, D), lambda i: (i, 0)),
                  wspec, wspec, wspec, wspec,
                  bspec, bspec, bspec, bspec],
        out_specs=[pl.BlockSpec((NBLK, D), lambda i: (i, 0))]
        + [qspec] * NQ + [kvspec] * NQ,
        out_shape=[jax.ShapeDtypeStruct((N, D), jnp.float32)]
        + [jax.ShapeDtypeStruct((N, QW), jnp.float32)] * NQ
        + [jax.ShapeDtypeStruct((N, 2 * QW), jnp.float32)] * NQ,
    )(x, W1, W2, W3, W4, b1, b2, b3, b4)


def _proj_edges_body(ea_ref, w5_ref, b5_ref, *o_refs):
    h5 = (jnp.dot(ea_ref[...], w5_ref[...],
                  preferred_element_type=jnp.float32) + b5_ref[...])
    for qi in range(NQ):
        o_refs[qi][...] = h5[:, QW * qi:QW * (qi + 1)]


def _proj_edges(edge_attr, W5, b5):
    grid = (E // EBLK,)
    return pl.pallas_call(
        _proj_edges_body,
        grid=grid,
        in_specs=[pl.BlockSpec((EBLK, D), lambda i: (i, 0)),
                  pl.BlockSpec((D, D), lambda i: (0, 0)),
                  pl.BlockSpec((1, D), lambda i: (0, 0))],
        out_specs=[pl.BlockSpec((EBLK, QW), lambda i: (i, 0))] * NQ,
        out_shape=[jax.ShapeDtypeStruct((E, QW), jnp.float32)] * NQ,
    )(edge_attr, W5, b5)


def _sc_body(dst_hbm, src_hbm,
             q0, kv0, w50, q1, kv1, w51, q2, kv2, w52, q3, kv3, w53,
             out_hbm,
             idxd0, idxd1, idxd2, idxd3, idxs0, idxs1, idxs2, idxs3,
             qv0, qv1, kvv0, kvv1, wv0, wv1, uv0, uv1, acc_sh,
             semg0, semg1, semi):
    idxd = [idxd0, idxd1, idxd2, idxd3]
    idxs = [idxs0, idxs1, idxs2, idxs3]
    qv = [qv0, qv1]
    kvv = [kvv0, kvv1]
    wv = [wv0, wv1]
    uv = [uv0, uv1]
    semg = [semg0, semg1]

    cid = lax.axis_index("c")
    sid = lax.axis_index("s")
    zeros16 = jnp.zeros((16,), jnp.float32)
    iota16 = lax.iota(jnp.int32, 16)

    base_row = sid * ROWS_PER_TILE
    t = cid * NS + sid
    start = t * CPT

    def run_pass(qi, q_hbm, kv_hbm, w5_hbm):
        for sl in range(2):
            uu = uv[sl]

            @pl.loop(0, C)
            def _z(r, uu=uu):
                for j in range(CW // 16):
                    uu[r, pl.ds(j * 16, 16)] = zeros16
                uu[r, pl.ds(CW - 16, 16)] = zeros16

        for j in range(ROWS_PER_TILE // C):
            pltpu.sync_copy(uv[0], acc_sh.at[pl.ds(base_row + j * C, C)])
        rem = ROWS_PER_TILE % C
        if rem:
            pltpu.sync_copy(
                uv[0].at[pl.ds(0, rem)],
                acc_sh.at[pl.ds(base_row + (ROWS_PER_TILE // C) * C, rem)])
        plsc.subcore_barrier()

        def issue_gathers(sl, il, base):
            wbase = jnp.minimum(base, E - C)
            pltpu.async_copy(q_hbm.at[idxd[il]], qv[sl], semg[sl])
            pltpu.async_copy(kv_hbm.at[idxs[il]], kvv[sl], semg[sl])
            pltpu.async_copy(w5_hbm.at[pl.ds(wbase, C)], wv[sl], semg[sl])

        def wait_gathers(sl, il, base):
            wbase = jnp.minimum(base, E - C)
            pltpu.make_async_copy(q_hbm.at[idxd[il]], qv[sl], semg[sl]).wait()
            pltpu.make_async_copy(kv_hbm.at[idxs[il]], kvv[sl],
                                  semg[sl]).wait()
            pltpu.make_async_copy(w5_hbm.at[pl.ds(wbase, C)], wv[sl],
                                  semg[sl]).wait()

        def compute(sl):
            uu = uv[sl]
            qq = qv[sl]
            kk = kvv[sl]
            ww = wv[sl]

            @plsc.parallel_loop(0, C, unroll=4)
            def _edge(r):
                qr = [qq[r, pl.ds(16 * i, 16)] for i in range(4)]
                kr = [kk[r, pl.ds(16 * i, 16)] for i in range(4)]
                vr = [kk[r, pl.ds(QW + 16 * i, 16)] for i in range(4)]
                wr = [ww[r, pl.ds(16 * i, 16)] for i in range(4)]
                ps = []
                for h in range(2):
                    th = (qr[2 * h] * (kr[2 * h] + wr[2 * h])
                          + qr[2 * h + 1] * (kr[2 * h + 1] + wr[2 * h + 1]))
                    s = jnp.sum(th) * INV_SQRT_HD
                    ps.append(jnp.exp(zeros16 + s))
                for i in range(4):
                    uu[r, pl.ds(16 * i, 16)] = (vr[i] + wr[i]) * ps[i // 2]
                pd = (jnp.where(iota16 == 0, ps[0], 0.0)
                      + jnp.where(iota16 == 1, ps[1], 0.0))
                plsc.store_scatter(uu, [iota16 * 0 + r, QW + (iota16 & 1)],
                                   pd, mask=iota16 < 2)

        pltpu.sync_copy(dst_hbm.at[pl.ds(start * C, C)], idxd[0])
        pltpu.sync_copy(src_hbm.at[pl.ds(start * C, C)], idxs[0])
        issue_gathers(0, 0, start * C)
        pltpu.async_copy(dst_hbm.at[pl.ds((start + 1) * C, C)], idxd[1], semi)
        pltpu.async_copy(src_hbm.at[pl.ds((start + 1) * C, C)], idxs[1], semi)

        @pl.loop(0, CPT // 4)
        def _quad(pi):
            for b in range(4):
                ci = 4 * pi + b
                s2 = b % 2
                n2 = (b + 1) % 2
                s4 = b
                n4 = (b + 1) % 4
                p4 = (b + 2) % 4
                base = (start + ci) * C
                nbase = base + C
                pbase = base + 2 * C

                pltpu.make_async_copy(dst_hbm.at[pl.ds(nbase, C)], idxd[n4],
                                      semi).wait()
                pltpu.make_async_copy(src_hbm.at[pl.ds(nbase, C)], idxs[n4],
                                      semi).wait()
                issue_gathers(n2, n4, nbase)

                pltpu.async_copy(dst_hbm.at[pl.ds(pbase, C)], idxd[p4], semi)
                pltpu.async_copy(src_hbm.at[pl.ds(pbase, C)], idxs[p4], semi)

                wait_gathers(s2, s4, base)
                compute(s2)
                pltpu.sync_copy(uv[s2], acc_sh.at[idxd[s4]], add=True)

        wait_gathers(0, 0, (start + CPT) * C)
        pltpu.make_async_copy(dst_hbm.at[pl.ds((start + CPT + 1) * C, C)],
                              idxd[1], semi).wait()
        pltpu.make_async_copy(src_hbm.at[pl.ds((start + CPT + 1) * C, C)],
                              idxs[1], semi).wait()

        plsc.subcore_barrier()
        pltpu.sync_copy(acc_sh.at[pl.ds(base_row, ROWS_PER_TILE)],
                        out_hbm.at[qi, cid, pl.ds(base_row, ROWS_PER_TILE)])

    run_pass(0, q0, kv0, w50)
    run_pass(1, q1, kv1, w51)
    run_pass(2, q2, kv2, w52)
    run_pass(3, q3, kv3, w53)


def _sc_edges(dst, src, qs, kvs, w5s):
    mesh = plsc.VectorSubcoreMesh(core_axis_name="c", subcore_axis_name="s",
                                  num_cores=NC, num_subcores=NS)
    f = pl.kernel(
        _sc_body,
        out_type=jax.ShapeDtypeStruct((NQ, NC, NT, CW), jnp.float32),
        mesh=mesh,
        compiler_params=pltpu.CompilerParams(use_tc_tiling_on_sc=False,
                                             needs_layout_passes=False),
        scratch_types=[
            pltpu.VMEM((C,), jnp.int32),
            pltpu.VMEM((C,), jnp.int32),
            pltpu.VMEM((C,), jnp.int32),
            pltpu.VMEM((C,), jnp.int32),
            pltpu.VMEM((C,), jnp.int32),
            pltpu.VMEM((C,), jnp.int32),
            pltpu.VMEM((C,), jnp.int32),
            pltpu.VMEM((C,), jnp.int32),
            pltpu.VMEM((C, QW), jnp.float32),
            pltpu.VMEM((C, QW), jnp.float32),
            pltpu.VMEM((C, 2 * QW), jnp.float32),
            pltpu.VMEM((C, 2 * QW), jnp.float32),
            pltpu.VMEM((C, QW), jnp.float32),
            pltpu.VMEM((C, QW), jnp.float32),
            pltpu.VMEM((C, CW), jnp.float32),
            pltpu.VMEM((C, CW), jnp.float32),
            pltpu.VMEM_SHARED((NT, CW), jnp.float32),
            pltpu.SemaphoreType.DMA,
            pltpu.SemaphoreType.DMA,
            pltpu.SemaphoreType.DMA,
        ],
    )
    return f(dst, src,
             qs[0], kvs[0], w5s[0], qs[1], kvs[1], w5s[1],
             qs[2], kvs[2], w5s[2], qs[3], kvs[3], w5s[3])


def _combine_body(w1_ref, p0_ref, p1_ref, p2_ref, p3_ref, out_ref):
    ri = lax.broadcasted_iota(jnp.int32, (2, QW), 0)
    ci = lax.broadcasted_iota(jnp.int32, (2, QW), 1)
    expand = (ci // HD == ri).astype(jnp.float32)
    outs = []
    for p_ref in (p0_ref, p1_ref, p2_ref, p3_ref):
        p = p_ref[0, 0] + p_ref[0, 1]
        agg = p[:, :QW]
        den = p[:, QW:QW + 2]
        recip = jnp.where(den != 0.0, 1.0 / den, 0.0)
        outs.append(agg * jnp.dot(recip, expand,
                                  preferred_element_type=jnp.float32))
    out_ref[...] = w1_ref[...] + jnp.concatenate(outs, axis=1)


def _combine(w1f, parts):
    grid = (N // NBLK,)

    def pspec(qi):
        return pl.BlockSpec((1, NC, NBLK, CW), lambda i, qi=qi: (qi, 0, i, 0))

    return pl.pallas_call(
        _combine_body,
        grid=grid,
        in_specs=[pl.BlockSpec((NBLK, D), lambda i: (i, 0)),
                  pspec(0), pspec(1), pspec(2), pspec(3)],
        out_specs=pl.BlockSpec((NBLK, D), lambda i: (i, 0)),
        out_shape=jax.ShapeDtypeStruct((N, D), jnp.float32),
    )(w1f, parts, parts, parts, parts)


def kernel(x, edge_index, edge_attr, W1, b1, W2, b2, W3, b3, W4, b4, W5, b5):
    b1r = b1.reshape(1, D)
    b2r = b2.reshape(1, D)
    b3r = b3.reshape(1, D)
    b4r = b4.reshape(1, D)
    b5r = b5.reshape(1, D)
    dst = jnp.concatenate(
        [edge_index[0], jnp.full((EPAD - E,), TRASH, dtype=jnp.int32)])
    src = jnp.concatenate(
        [edge_index[1], jnp.zeros((EPAD - E,), dtype=jnp.int32)])

    w1f, *nodes = _proj_nodes(x, W1, b1r, W2, b2r, W3, b3r, W4, b4r)
    zq = jnp.zeros((NT - N, QW), jnp.float32)
    qs = [jnp.concatenate([q, zq]) for q in nodes[:NQ]]
    kvs = nodes[NQ:]
    w5s = _proj_edges(edge_attr, W5, b5r)

    parts = _sc_edges(dst, src, qs, kvs, w5s)

    return _combine(w1f, parts)


# speedup vs baseline: 2.6891x; 1.7667x over previous
import math

import jax
import jax.numpy as jnp
from jax import lax
from jax.experimental import pallas as pl
from jax.experimental.pallas import tpu as pltpu
from jax.experimental.pallas import tpu_sc as plsc

N = 10000
E = 160000
D = 256
H = 8
HD = 32
HW = 128
NP = 2
NC = 2
NS = 16
NT = 10016
TRASH = 10000
ROWS_PER_TILE = NT // NS
CW = 136
C = 32
TILES = NC * NS
CPT = 160
EPAD = (TILES * CPT + 2) * C
INV_SQRT_HD = 1.0 / math.sqrt(HD)

NBLK = 1000
EBLK = 1000


def _proj_nodes_body(x_ref, w1_ref, w2_ref, w3_ref, w4_ref,
                     b1_ref, b2_ref, b3_ref, b4_ref,
                     o1_ref, *oq_refs):
    xb = x_ref[...]
    o1_ref[...] = (jnp.dot(xb, w1_ref[...], preferred_element_type=jnp.float32)
                   + b1_ref[...])
    h3 = jnp.dot(xb, w3_ref[...], preferred_element_type=jnp.float32) + b3_ref[...]
    h4 = jnp.dot(xb, w4_ref[...], preferred_element_type=jnp.float32) + b4_ref[...]
    h2 = jnp.dot(xb, w2_ref[...], preferred_element_type=jnp.float32) + b2_ref[...]
    for qi in range(NP):
        sl = slice(HW * qi, HW * (qi + 1))
        oq_refs[qi][...] = h3[:, sl]
        oq_refs[NP + qi][...] = jnp.concatenate([h4[:, sl], h2[:, sl]], axis=1)


def _proj_nodes(x, W1, b1, W2, b2, W3, b3, W4, b4):
    grid = (N // NBLK,)
    wspec = pl.BlockSpec((D, D), lambda i: (0, 0))
    bspec = pl.BlockSpec((1, D), lambda i: (0, 0))
    qspec = pl.BlockSpec((NBLK, HW), lambda i: (i, 0))
    kvspec = pl.BlockSpec((NBLK, 2 * HW), lambda i: (i, 0))
    return pl.pallas_call(
        _proj_nodes_body,
        grid=grid,
        in_specs=[pl.BlockSpec((NBLK, D), lambda i: (i, 0)),
                  wspec, wspec, wspec, wspec,
                  bspec, bspec, bspec, bspec],
        out_specs=[pl.BlockSpec((NBLK, D), lambda i: (i, 0))]
        + [qspec] * NP + [kvspec] * NP,
        out_shape=[jax.ShapeDtypeStruct((N, D), jnp.float32)]
        + [jax.ShapeDtypeStruct((N, HW), jnp.float32)] * NP
        + [jax.ShapeDtypeStruct((N, 2 * HW), jnp.float32)] * NP,
    )(x, W1, W2, W3, W4, b1, b2, b3, b4)


def _proj_edges_body(ea_ref, w5_ref, b5_ref, *o_refs):
    h5 = (jnp.dot(ea_ref[...], w5_ref[...],
                  preferred_element_type=jnp.float32) + b5_ref[...])
    for qi in range(NP):
        o_refs[qi][...] = h5[:, HW * qi:HW * (qi + 1)]


def _proj_edges(edge_attr, W5, b5):
    grid = (E // EBLK,)
    return pl.pallas_call(
        _proj_edges_body,
        grid=grid,
        in_specs=[pl.BlockSpec((EBLK, D), lambda i: (i, 0)),
                  pl.BlockSpec((D, D), lambda i: (0, 0)),
                  pl.BlockSpec((1, D), lambda i: (0, 0))],
        out_specs=[pl.BlockSpec((EBLK, HW), lambda i: (i, 0))] * NP,
        out_shape=[jax.ShapeDtypeStruct((E, HW), jnp.float32)] * NP,
    )(edge_attr, W5, b5)


def _sc_body(dst_hbm, src_hbm,
             q0, kv0, w50, q1, kv1, w51,
             out_hbm,
             idxd0, idxd1, idxd2, idxd3, idxs0, idxs1, idxs2, idxs3,
             qv0, qv1, kvv0, kvv1, wv0, wv1, uv0, uv1, acc_sh,
             semg0, semg1, semi):
    idxd = [idxd0, idxd1, idxd2, idxd3]
    idxs = [idxs0, idxs1, idxs2, idxs3]
    qv = [qv0, qv1]
    kvv = [kvv0, kvv1]
    wv = [wv0, wv1]
    uv = [uv0, uv1]
    semg = [semg0, semg1]

    cid = lax.axis_index("c")
    sid = lax.axis_index("s")
    zeros16 = jnp.zeros((16,), jnp.float32)
    iota16 = lax.iota(jnp.int32, 16)

    base_row = sid * ROWS_PER_TILE
    t = cid * NS + sid
    start = t * CPT

    def run_pass(qi, q_hbm, kv_hbm, w5_hbm):
        for sl in range(2):
            uu = uv[sl]

            @pl.loop(0, C)
            def _z(r, uu=uu):
                for j in range(CW // 16):
                    uu[r, pl.ds(j * 16, 16)] = zeros16
                uu[r, pl.ds(CW - 16, 16)] = zeros16

        for j in range(ROWS_PER_TILE // C):
            pltpu.sync_copy(uv[0], acc_sh.at[pl.ds(base_row + j * C, C)])
        rem = ROWS_PER_TILE % C
        if rem:
            pltpu.sync_copy(
                uv[0].at[pl.ds(0, rem)],
                acc_sh.at[pl.ds(base_row + (ROWS_PER_TILE // C) * C, rem)])
        plsc.subcore_barrier()

        def issue_gathers(sl, il, base):
            wbase = jnp.minimum(base, E - C)
            pltpu.async_copy(q_hbm.at[idxd[il]], qv[sl], semg[sl])
            pltpu.async_copy(kv_hbm.at[idxs[il]], kvv[sl], semg[sl])
            pltpu.async_copy(w5_hbm.at[pl.ds(wbase, C)], wv[sl], semg[sl])

        def wait_gathers(sl, il, base):
            wbase = jnp.minimum(base, E - C)
            pltpu.make_async_copy(q_hbm.at[idxd[il]], qv[sl], semg[sl]).wait()
            pltpu.make_async_copy(kv_hbm.at[idxs[il]], kvv[sl],
                                  semg[sl]).wait()
            pltpu.make_async_copy(w5_hbm.at[pl.ds(wbase, C)], wv[sl],
                                  semg[sl]).wait()

        def compute(sl):
            uu = uv[sl]
            qq = qv[sl]
            kk = kvv[sl]
            ww = wv[sl]

            @plsc.parallel_loop(0, C, unroll=2)
            def _edge(r):
                qr = [qq[r, pl.ds(16 * i, 16)] for i in range(8)]
                kr = [kk[r, pl.ds(16 * i, 16)] for i in range(8)]
                vr = [kk[r, pl.ds(HW + 16 * i, 16)] for i in range(8)]
                wr = [ww[r, pl.ds(16 * i, 16)] for i in range(8)]
                ps = []
                for h in range(4):
                    th = (qr[2 * h] * (kr[2 * h] + wr[2 * h])
                          + qr[2 * h + 1] * (kr[2 * h + 1] + wr[2 * h + 1]))
                    s = jnp.sum(th) * INV_SQRT_HD
                    ps.append(jnp.exp(zeros16 + s))
                for i in range(8):
                    uu[r, pl.ds(16 * i, 16)] = (vr[i] + wr[i]) * ps[i // 2]
                pd = (jnp.where(iota16 == 0, ps[0], 0.0)
                      + jnp.where(iota16 == 1, ps[1], 0.0)
                      + jnp.where(iota16 == 2, ps[2], 0.0)
                      + jnp.where(iota16 == 3, ps[3], 0.0))
                plsc.store_scatter(uu, [iota16 * 0 + r, HW + (iota16 & 3)],
                                   pd, mask=iota16 < 4)

        pltpu.sync_copy(dst_hbm.at[pl.ds(start * C, C)], idxd[0])
        pltpu.sync_copy(src_hbm.at[pl.ds(start * C, C)], idxs[0])
        issue_gathers(0, 0, start * C)
        pltpu.async_copy(dst_hbm.at[pl.ds((start + 1) * C, C)], idxd[1], semi)
        pltpu.async_copy(src_hbm.at[pl.ds((start + 1) * C, C)], idxs[1], semi)

        @pl.loop(0, CPT // 4)
        def _quad(pi):
            for b in range(4):
                ci = 4 * pi + b
                s2 = b % 2
                n2 = (b + 1) % 2
                s4 = b
                n4 = (b + 1) % 4
                p4 = (b + 2) % 4
                base = (start + ci) * C
                nbase = base + C
                pbase = base + 2 * C

                pltpu.make_async_copy(dst_hbm.at[pl.ds(nbase, C)], idxd[n4],
                                      semi).wait()
                pltpu.make_async_copy(src_hbm.at[pl.ds(nbase, C)], idxs[n4],
                                      semi).wait()
                issue_gathers(n2, n4, nbase)

                pltpu.async_copy(dst_hbm.at[pl.ds(pbase, C)], idxd[p4], semi)
                pltpu.async_copy(src_hbm.at[pl.ds(pbase, C)], idxs[p4], semi)

                wait_gathers(s2, s4, base)
                compute(s2)
                pltpu.sync_copy(uv[s2], acc_sh.at[idxd[s4]], add=True)

        wait_gathers(0, 0, (start + CPT) * C)
        pltpu.make_async_copy(dst_hbm.at[pl.ds((start + CPT + 1) * C, C)],
                              idxd[1], semi).wait()
        pltpu.make_async_copy(src_hbm.at[pl.ds((start + CPT + 1) * C, C)],
                              idxs[1], semi).wait()

        plsc.subcore_barrier()
        pltpu.sync_copy(acc_sh.at[pl.ds(base_row, ROWS_PER_TILE)],
                        out_hbm.at[qi, cid, pl.ds(base_row, ROWS_PER_TILE)])

    run_pass(0, q0, kv0, w50)
    run_pass(1, q1, kv1, w51)


def _sc_edges(dst, src, qs, kvs, w5s):
    mesh = plsc.VectorSubcoreMesh(core_axis_name="c", subcore_axis_name="s",
                                  num_cores=NC, num_subcores=NS)
    f = pl.kernel(
        _sc_body,
        out_type=jax.ShapeDtypeStruct((NP, NC, NT, CW), jnp.float32),
        mesh=mesh,
        compiler_params=pltpu.CompilerParams(use_tc_tiling_on_sc=False,
                                             needs_layout_passes=False),
        scratch_types=[
            pltpu.VMEM((C,), jnp.int32),
            pltpu.VMEM((C,), jnp.int32),
            pltpu.VMEM((C,), jnp.int32),
            pltpu.VMEM((C,), jnp.int32),
            pltpu.VMEM((C,), jnp.int32),
            pltpu.VMEM((C,), jnp.int32),
            pltpu.VMEM((C,), jnp.int32),
            pltpu.VMEM((C,), jnp.int32),
            pltpu.VMEM((C, HW), jnp.float32),
            pltpu.VMEM((C, HW), jnp.float32),
            pltpu.VMEM((C, 2 * HW), jnp.float32),
            pltpu.VMEM((C, 2 * HW), jnp.float32),
            pltpu.VMEM((C, HW), jnp.float32),
            pltpu.VMEM((C, HW), jnp.float32),
            pltpu.VMEM((C, CW), jnp.float32),
            pltpu.VMEM((C, CW), jnp.float32),
            pltpu.VMEM_SHARED((NT, CW), jnp.float32),
            pltpu.SemaphoreType.DMA,
            pltpu.SemaphoreType.DMA,
            pltpu.SemaphoreType.DMA,
        ],
    )
    return f(dst, src, qs[0], kvs[0], w5s[0], qs[1], kvs[1], w5s[1])


def _combine_body(w1_ref, p0_ref, p1_ref, out_ref):
    ri = lax.broadcasted_iota(jnp.int32, (4, HW), 0)
    ci = lax.broadcasted_iota(jnp.int32, (4, HW), 1)
    expand = (ci // HD == ri).astype(jnp.float32)
    outs = []
    for p_ref in (p0_ref, p1_ref):
        p = p_ref[0, 0] + p_ref[0, 1]
        agg = p[:, :HW]
        den = p[:, HW:HW + 4]
        recip = jnp.where(den != 0.0, 1.0 / den, 0.0)
        outs.append(agg * jnp.dot(recip, expand,
                                  preferred_element_type=jnp.float32))
    out_ref[...] = w1_ref[...] + jnp.concatenate(outs, axis=1)


def _combine(w1f, parts):
    grid = (N // NBLK,)

    def pspec(qi):
        return pl.BlockSpec((1, NC, NBLK, CW), lambda i, qi=qi: (qi, 0, i, 0))

    return pl.pallas_call(
        _combine_body,
        grid=grid,
        in_specs=[pl.BlockSpec((NBLK, D), lambda i: (i, 0)),
                  pspec(0), pspec(1)],
        out_specs=pl.BlockSpec((NBLK, D), lambda i: (i, 0)),
        out_shape=jax.ShapeDtypeStruct((N, D), jnp.float32),
    )(w1f, parts, parts)


def kernel(x, edge_index, edge_attr, W1, b1, W2, b2, W3, b3, W4, b4, W5, b5):
    b1r = b1.reshape(1, D)
    b2r = b2.reshape(1, D)
    b3r = b3.reshape(1, D)
    b4r = b4.reshape(1, D)
    b5r = b5.reshape(1, D)
    dst = jnp.concatenate(
        [edge_index[0], jnp.full((EPAD - E,), TRASH, dtype=jnp.int32)])
    src = jnp.concatenate(
        [edge_index[1], jnp.zeros((EPAD - E,), dtype=jnp.int32)])

    w1f, *nodes = _proj_nodes(x, W1, b1r, W2, b2r, W3, b3r, W4, b4r)
    zq = jnp.zeros((NT - N, HW), jnp.float32)
    qs = [jnp.concatenate([q, zq]) for q in nodes[:NP]]
    kvs = nodes[NP:]
    w5s = _proj_edges(edge_attr, W5, b5r)

    parts = _sc_edges(dst, src, qs, kvs, w5s)

    return _combine(w1f, parts)


# speedup vs baseline: 2.7114x; 1.0083x over previous
import math

import jax
import jax.numpy as jnp
from jax import lax
from jax.experimental import pallas as pl
from jax.experimental.pallas import tpu as pltpu
from jax.experimental.pallas import tpu_sc as plsc

N = 10000
E = 160000
D = 256
H = 8
HD = 32
HW = 128
NP = 2
NC = 2
NS = 16
NT = 10016
TRASH = 10000
ROWS_PER_TILE = NT // NS
CW = 136
C = 32
TILES = NC * NS
CPT = 160
EPAD = (TILES * CPT + 2) * C
INV_SQRT_HD = 1.0 / math.sqrt(HD)

NBLK = 1000
EBLK = 1000


def _proj_nodes_body(x_ref, w1_ref, w2_ref, w3_ref, w4_ref,
                     b1_ref, b2_ref, b3_ref, b4_ref,
                     o1_ref, *oq_refs):
    xb = x_ref[...]
    o1_ref[...] = (jnp.dot(xb, w1_ref[...], preferred_element_type=jnp.float32)
                   + b1_ref[...])
    h3 = jnp.dot(xb, w3_ref[...], preferred_element_type=jnp.float32) + b3_ref[...]
    h4 = jnp.dot(xb, w4_ref[...], preferred_element_type=jnp.float32) + b4_ref[...]
    h2 = jnp.dot(xb, w2_ref[...], preferred_element_type=jnp.float32) + b2_ref[...]
    for qi in range(NP):
        sl = slice(HW * qi, HW * (qi + 1))
        oq_refs[qi][...] = h3[:, sl]
        oq_refs[NP + qi][...] = jnp.concatenate([h4[:, sl], h2[:, sl]], axis=1)


def _proj_nodes(x, W1, b1, W2, b2, W3, b3, W4, b4):
    grid = (N // NBLK,)
    wspec = pl.BlockSpec((D, D), lambda i: (0, 0))
    bspec = pl.BlockSpec((1, D), lambda i: (0, 0))
    qspec = pl.BlockSpec((NBLK, HW), lambda i: (i, 0))
    kvspec = pl.BlockSpec((NBLK, 2 * HW), lambda i: (i, 0))
    return pl.pallas_call(
        _proj_nodes_body,
        grid=grid,
        in_specs=[pl.BlockSpec((NBLK, D), lambda i: (i, 0)),
                  wspec, wspec, wspec, wspec,
                  bspec, bspec, bspec, bspec],
        out_specs=[pl.BlockSpec((NBLK, D), lambda i: (i, 0))]
        + [qspec] * NP + [kvspec] * NP,
        out_shape=[jax.ShapeDtypeStruct((N, D), jnp.float32)]
        + [jax.ShapeDtypeStruct((N, HW), jnp.float32)] * NP
        + [jax.ShapeDtypeStruct((N, 2 * HW), jnp.float32)] * NP,
    )(x, W1, W2, W3, W4, b1, b2, b3, b4)


def _proj_edges_body(ea_ref, w5_ref, b5_ref, *o_refs):
    h5 = (jnp.dot(ea_ref[...], w5_ref[...],
                  preferred_element_type=jnp.float32) + b5_ref[...])
    for qi in range(NP):
        o_refs[qi][...] = h5[:, HW * qi:HW * (qi + 1)]


def _proj_edges(edge_attr, W5, b5):
    grid = (E // EBLK,)
    return pl.pallas_call(
        _proj_edges_body,
        grid=grid,
        in_specs=[pl.BlockSpec((EBLK, D), lambda i: (i, 0)),
                  pl.BlockSpec((D, D), lambda i: (0, 0)),
                  pl.BlockSpec((1, D), lambda i: (0, 0))],
        out_specs=[pl.BlockSpec((EBLK, HW), lambda i: (i, 0))] * NP,
        out_shape=[jax.ShapeDtypeStruct((E, HW), jnp.float32)] * NP,
    )(edge_attr, W5, b5)


def _sc_body(dst_hbm, src_hbm,
             q0, kv0, w50, q1, kv1, w51,
             out_hbm,
             idxd0, idxd1, idxd2, idxd3, idxs0, idxs1, idxs2, idxs3,
             qv0, qv1, kvv0, kvv1, wv0, wv1, uv0, uv1, acc_sh,
             semg0, semg1, semi):
    idxd = [idxd0, idxd1, idxd2, idxd3]
    idxs = [idxs0, idxs1, idxs2, idxs3]
    qv = [qv0, qv1]
    kvv = [kvv0, kvv1]
    wv = [wv0, wv1]
    uv = [uv0, uv1]
    semg = [semg0, semg1]

    cid = lax.axis_index("c")
    sid = lax.axis_index("s")
    zeros16 = jnp.zeros((16,), jnp.float32)
    iota16 = lax.iota(jnp.int32, 16)

    base_row = sid * ROWS_PER_TILE
    t = cid * NS + sid
    start = t * CPT

    def run_pass(qi, q_hbm, kv_hbm, w5_hbm):
        for sl in range(2):
            uu = uv[sl]

            @pl.loop(0, C)
            def _z(r, uu=uu):
                for j in range(CW // 16):
                    uu[r, pl.ds(j * 16, 16)] = zeros16
                uu[r, pl.ds(CW - 16, 16)] = zeros16

        for j in range(ROWS_PER_TILE // C):
            pltpu.sync_copy(uv[0], acc_sh.at[pl.ds(base_row + j * C, C)])
        rem = ROWS_PER_TILE % C
        if rem:
            pltpu.sync_copy(
                uv[0].at[pl.ds(0, rem)],
                acc_sh.at[pl.ds(base_row + (ROWS_PER_TILE // C) * C, rem)])
        plsc.subcore_barrier()

        def issue_gathers(sl, il, base):
            wbase = jnp.minimum(base, E - C)
            pltpu.async_copy(q_hbm.at[idxd[il]], qv[sl], semg[sl])
            pltpu.async_copy(kv_hbm.at[idxs[il]], kvv[sl], semg[sl])
            pltpu.async_copy(w5_hbm.at[pl.ds(wbase, C)], wv[sl], semg[sl])

        def wait_gathers(sl, il, base):
            wbase = jnp.minimum(base, E - C)
            pltpu.make_async_copy(q_hbm.at[idxd[il]], qv[sl], semg[sl]).wait()
            pltpu.make_async_copy(kv_hbm.at[idxs[il]], kvv[sl],
                                  semg[sl]).wait()
            pltpu.make_async_copy(w5_hbm.at[pl.ds(wbase, C)], wv[sl],
                                  semg[sl]).wait()

        def compute(sl):
            uu = uv[sl]
            qq = qv[sl]
            kk = kvv[sl]
            ww = wv[sl]

            @plsc.parallel_loop(0, C, unroll=4)
            def _edge(r):
                qr = [qq[r, pl.ds(16 * i, 16)] for i in range(8)]
                kr = [kk[r, pl.ds(16 * i, 16)] for i in range(8)]
                vr = [kk[r, pl.ds(HW + 16 * i, 16)] for i in range(8)]
                wr = [ww[r, pl.ds(16 * i, 16)] for i in range(8)]
                ps = []
                for h in range(4):
                    th = (qr[2 * h] * (kr[2 * h] + wr[2 * h])
                          + qr[2 * h + 1] * (kr[2 * h + 1] + wr[2 * h + 1]))
                    s = jnp.sum(th) * INV_SQRT_HD
                    ps.append(jnp.exp(zeros16 + s))
                for i in range(8):
                    uu[r, pl.ds(16 * i, 16)] = (vr[i] + wr[i]) * ps[i // 2]
                pd = (jnp.where(iota16 == 0, ps[0], 0.0)
                      + jnp.where(iota16 == 1, ps[1], 0.0)
                      + jnp.where(iota16 == 2, ps[2], 0.0)
                      + jnp.where(iota16 == 3, ps[3], 0.0))
                plsc.store_scatter(uu, [iota16 * 0 + r, HW + (iota16 & 3)],
                                   pd, mask=iota16 < 4)

        pltpu.sync_copy(dst_hbm.at[pl.ds(start * C, C)], idxd[0])
        pltpu.sync_copy(src_hbm.at[pl.ds(start * C, C)], idxs[0])
        issue_gathers(0, 0, start * C)
        pltpu.async_copy(dst_hbm.at[pl.ds((start + 1) * C, C)], idxd[1], semi)
        pltpu.async_copy(src_hbm.at[pl.ds((start + 1) * C, C)], idxs[1], semi)

        @pl.loop(0, CPT // 4)
        def _quad(pi):
            for b in range(4):
                ci = 4 * pi + b
                s2 = b % 2
                n2 = (b + 1) % 2
                s4 = b
                n4 = (b + 1) % 4
                p4 = (b + 2) % 4
                base = (start + ci) * C
                nbase = base + C
                pbase = base + 2 * C

                pltpu.make_async_copy(dst_hbm.at[pl.ds(nbase, C)], idxd[n4],
                                      semi).wait()
                pltpu.make_async_copy(src_hbm.at[pl.ds(nbase, C)], idxs[n4],
                                      semi).wait()
                issue_gathers(n2, n4, nbase)

                pltpu.async_copy(dst_hbm.at[pl.ds(pbase, C)], idxd[p4], semi)
                pltpu.async_copy(src_hbm.at[pl.ds(pbase, C)], idxs[p4], semi)

                wait_gathers(s2, s4, base)
                compute(s2)
                pltpu.sync_copy(uv[s2], acc_sh.at[idxd[s4]], add=True)

        wait_gathers(0, 0, (start + CPT) * C)
        pltpu.make_async_copy(dst_hbm.at[pl.ds((start + CPT + 1) * C, C)],
                              idxd[1], semi).wait()
        pltpu.make_async_copy(src_hbm.at[pl.ds((start + CPT + 1) * C, C)],
                              idxs[1], semi).wait()

        plsc.subcore_barrier()
        pltpu.sync_copy(acc_sh.at[pl.ds(base_row, ROWS_PER_TILE)],
                        out_hbm.at[qi, cid, pl.ds(base_row, ROWS_PER_TILE)])

    run_pass(0, q0, kv0, w50)
    run_pass(1, q1, kv1, w51)


def _sc_edges(dst, src, qs, kvs, w5s):
    mesh = plsc.VectorSubcoreMesh(core_axis_name="c", subcore_axis_name="s",
                                  num_cores=NC, num_subcores=NS)
    f = pl.kernel(
        _sc_body,
        out_type=jax.ShapeDtypeStruct((NP, NC, NT, CW), jnp.float32),
        mesh=mesh,
        compiler_params=pltpu.CompilerParams(use_tc_tiling_on_sc=False,
                                             needs_layout_passes=False),
        scratch_types=[
            pltpu.VMEM((C,), jnp.int32),
            pltpu.VMEM((C,), jnp.int32),
            pltpu.VMEM((C,), jnp.int32),
            pltpu.VMEM((C,), jnp.int32),
            pltpu.VMEM((C,), jnp.int32),
            pltpu.VMEM((C,), jnp.int32),
            pltpu.VMEM((C,), jnp.int32),
            pltpu.VMEM((C,), jnp.int32),
            pltpu.VMEM((C, HW), jnp.float32),
            pltpu.VMEM((C, HW), jnp.float32),
            pltpu.VMEM((C, 2 * HW), jnp.float32),
            pltpu.VMEM((C, 2 * HW), jnp.float32),
            pltpu.VMEM((C, HW), jnp.float32),
            pltpu.VMEM((C, HW), jnp.float32),
            pltpu.VMEM((C, CW), jnp.float32),
            pltpu.VMEM((C, CW), jnp.float32),
            pltpu.VMEM_SHARED((NT, CW), jnp.float32),
            pltpu.SemaphoreType.DMA,
            pltpu.SemaphoreType.DMA,
            pltpu.SemaphoreType.DMA,
        ],
    )
    return f(dst, src, qs[0], kvs[0], w5s[0], qs[1], kvs[1], w5s[1])


def _combine_body(w1_ref, p0_ref, p1_ref, out_ref):
    ri = lax.broadcasted_iota(jnp.int32, (4, HW), 0)
    ci = lax.broadcasted_iota(jnp.int32, (4, HW), 1)
    expand = (ci // HD == ri).astype(jnp.float32)
    outs = []
    for p_ref in (p0_ref, p1_ref):
        p = p_ref[0, 0] + p_ref[0, 1]
        agg = p[:, :HW]
        den = p[:, HW:HW + 4]
        recip = jnp.where(den != 0.0, 1.0 / den, 0.0)
        outs.append(agg * jnp.dot(recip, expand,
                                  preferred_element_type=jnp.float32))
    out_ref[...] = w1_ref[...] + jnp.concatenate(outs, axis=1)


def _combine(w1f, parts):
    grid = (N // NBLK,)

    def pspec(qi):
        return pl.BlockSpec((1, NC, NBLK, CW), lambda i, qi=qi: (qi, 0, i, 0))

    return pl.pallas_call(
        _combine_body,
        grid=grid,
        in_specs=[pl.BlockSpec((NBLK, D), lambda i: (i, 0)),
                  pspec(0), pspec(1)],
        out_specs=pl.BlockSpec((NBLK, D), lambda i: (i, 0)),
        out_shape=jax.ShapeDtypeStruct((N, D), jnp.float32),
    )(w1f, parts, parts)


def kernel(x, edge_index, edge_attr, W1, b1, W2, b2, W3, b3, W4, b4, W5, b5):
    b1r = b1.reshape(1, D)
    b2r = b2.reshape(1, D)
    b3r = b3.reshape(1, D)
    b4r = b4.reshape(1, D)
    b5r = b5.reshape(1, D)
    dst = jnp.concatenate(
        [edge_index[0],
         TRASH + (jnp.arange(EPAD - E, dtype=jnp.int32) % (NT - N))])
    src = jnp.concatenate(
        [edge_index[1], jnp.zeros((EPAD - E,), dtype=jnp.int32)])

    w1f, *nodes = _proj_nodes(x, W1, b1r, W2, b2r, W3, b3r, W4, b4r)
    zq = jnp.zeros((NT - N, HW), jnp.float32)
    qs = [jnp.concatenate([q, zq]) for q in nodes[:NP]]
    kvs = nodes[NP:]
    w5s = _proj_edges(edge_attr, W5, b5r)

    parts = _sc_edges(dst, src, qs, kvs, w5s)

    return _combine(w1f, parts)


# speedup vs baseline: 2.7238x; 1.0046x over previous
import math

import jax
import jax.numpy as jnp
from jax import lax
from jax.experimental import pallas as pl
from jax.experimental.pallas import tpu as pltpu
from jax.experimental.pallas import tpu_sc as plsc

N = 10000
E = 160000
D = 256
H = 8
HD = 32
HW = 128
NP = 2
NC = 2
NS = 16
NT = 10016
TRASH = 10000
ROWS_PER_TILE = NT // NS
CW = 136
C = 32
TILES = NC * NS
CPT = 160
EPAD = (TILES * CPT + 2) * C
INV_SQRT_HD = 1.0 / math.sqrt(HD)

NBLK = 1000
EBLK = 1000


def _proj_nodes_body(x_ref, w1_ref, w2_ref, w3_ref, w4_ref,
                     b1_ref, b2_ref, b3_ref, b4_ref,
                     o1_ref, *oq_refs):
    xb = x_ref[...]
    o1_ref[...] = (jnp.dot(xb, w1_ref[...], preferred_element_type=jnp.float32)
                   + b1_ref[...])
    h3 = jnp.dot(xb, w3_ref[...], preferred_element_type=jnp.float32) + b3_ref[...]
    h4 = jnp.dot(xb, w4_ref[...], preferred_element_type=jnp.float32) + b4_ref[...]
    h2 = jnp.dot(xb, w2_ref[...], preferred_element_type=jnp.float32) + b2_ref[...]
    for qi in range(NP):
        sl = slice(HW * qi, HW * (qi + 1))
        oq_refs[qi][...] = h3[:, sl]
        oq_refs[NP + qi][...] = jnp.concatenate([h4[:, sl], h2[:, sl]], axis=1)


def _proj_nodes(x, W1, b1, W2, b2, W3, b3, W4, b4):
    grid = (N // NBLK,)
    wspec = pl.BlockSpec((D, D), lambda i: (0, 0))
    bspec = pl.BlockSpec((1, D), lambda i: (0, 0))
    qspec = pl.BlockSpec((NBLK, HW), lambda i: (i, 0))
    kvspec = pl.BlockSpec((NBLK, 2 * HW), lambda i: (i, 0))
    return pl.pallas_call(
        _proj_nodes_body,
        grid=grid,
        in_specs=[pl.BlockSpec((NBLK, D), lambda i: (i, 0)),
                  wspec, wspec, wspec, wspec,
                  bspec, bspec, bspec, bspec],
        out_specs=[pl.BlockSpec((NBLK, D), lambda i: (i, 0))]
        + [qspec] * NP + [kvspec] * NP,
        out_shape=[jax.ShapeDtypeStruct((N, D), jnp.float32)]
        + [jax.ShapeDtypeStruct((N, HW), jnp.float32)] * NP
        + [jax.ShapeDtypeStruct((N, 2 * HW), jnp.float32)] * NP,
    )(x, W1, W2, W3, W4, b1, b2, b3, b4)


def _proj_edges_body(ea_ref, w5_ref, b5_ref, *o_refs):
    h5 = (jnp.dot(ea_ref[...], w5_ref[...],
                  preferred_element_type=jnp.float32) + b5_ref[...])
    for qi in range(NP):
        o_refs[qi][...] = h5[:, HW * qi:HW * (qi + 1)]


def _proj_edges(edge_attr, W5, b5):
    grid = (E // EBLK,)
    return pl.pallas_call(
        _proj_edges_body,
        grid=grid,
        in_specs=[pl.BlockSpec((EBLK, D), lambda i: (i, 0)),
                  pl.BlockSpec((D, D), lambda i: (0, 0)),
                  pl.BlockSpec((1, D), lambda i: (0, 0))],
        out_specs=[pl.BlockSpec((EBLK, HW), lambda i: (i, 0))] * NP,
        out_shape=[jax.ShapeDtypeStruct((E, HW), jnp.float32)] * NP,
    )(edge_attr, W5, b5)


def _sc_body(dst_hbm, src_hbm,
             q0, kv0, w50, q1, kv1, w51,
             out_hbm,
             idxd0, idxd1, idxd2, idxd3, idxs0, idxs1, idxs2, idxs3,
             qv0, qv1, kvv0, kvv1, wv0, wv1, uv0, uv1, acc_sh,
             semg0, semg1, semsc0, semsc1, semi):
    idxd = [idxd0, idxd1, idxd2, idxd3]
    idxs = [idxs0, idxs1, idxs2, idxs3]
    qv = [qv0, qv1]
    kvv = [kvv0, kvv1]
    wv = [wv0, wv1]
    uv = [uv0, uv1]
    semg = [semg0, semg1]
    semsc = [semsc0, semsc1]

    cid = lax.axis_index("c")
    sid = lax.axis_index("s")
    zeros16 = jnp.zeros((16,), jnp.float32)
    iota16 = lax.iota(jnp.int32, 16)

    base_row = sid * ROWS_PER_TILE
    t = cid * NS + sid
    start = t * CPT

    def run_pass(qi, q_hbm, kv_hbm, w5_hbm):
        for sl in range(2):
            uu = uv[sl]

            @pl.loop(0, C)
            def _z(r, uu=uu):
                for j in range(CW // 16):
                    uu[r, pl.ds(j * 16, 16)] = zeros16
                uu[r, pl.ds(CW - 16, 16)] = zeros16

        for j in range(ROWS_PER_TILE // C):
            pltpu.sync_copy(uv[0], acc_sh.at[pl.ds(base_row + j * C, C)])
        rem = ROWS_PER_TILE % C
        if rem:
            pltpu.sync_copy(
                uv[0].at[pl.ds(0, rem)],
                acc_sh.at[pl.ds(base_row + (ROWS_PER_TILE // C) * C, rem)])
        plsc.subcore_barrier()

        def issue_gathers(sl, il, base):
            wbase = jnp.minimum(base, E - C)
            pltpu.async_copy(q_hbm.at[idxd[il]], qv[sl], semg[sl])
            pltpu.async_copy(kv_hbm.at[idxs[il]], kvv[sl], semg[sl])
            pltpu.async_copy(w5_hbm.at[pl.ds(wbase, C)], wv[sl], semg[sl])

        def wait_gathers(sl, il, base):
            wbase = jnp.minimum(base, E - C)
            pltpu.make_async_copy(q_hbm.at[idxd[il]], qv[sl], semg[sl]).wait()
            pltpu.make_async_copy(kv_hbm.at[idxs[il]], kvv[sl],
                                  semg[sl]).wait()
            pltpu.make_async_copy(w5_hbm.at[pl.ds(wbase, C)], wv[sl],
                                  semg[sl]).wait()

        def compute(sl):
            uu = uv[sl]
            qq = qv[sl]
            kk = kvv[sl]
            ww = wv[sl]

            @plsc.parallel_loop(0, C, unroll=4)
            def _edge(r):
                qr = [qq[r, pl.ds(16 * i, 16)] for i in range(8)]
                kr = [kk[r, pl.ds(16 * i, 16)] for i in range(8)]
                vr = [kk[r, pl.ds(HW + 16 * i, 16)] for i in range(8)]
                wr = [ww[r, pl.ds(16 * i, 16)] for i in range(8)]
                ps = []
                for h in range(4):
                    th = (qr[2 * h] * (kr[2 * h] + wr[2 * h])
                          + qr[2 * h + 1] * (kr[2 * h + 1] + wr[2 * h + 1]))
                    s = jnp.sum(th) * INV_SQRT_HD
                    ps.append(jnp.exp(zeros16 + s))
                for i in range(8):
                    uu[r, pl.ds(16 * i, 16)] = (vr[i] + wr[i]) * ps[i // 2]
                pd = (jnp.where(iota16 == 0, ps[0], 0.0)
                      + jnp.where(iota16 == 1, ps[1], 0.0)
                      + jnp.where(iota16 == 2, ps[2], 0.0)
                      + jnp.where(iota16 == 3, ps[3], 0.0))
                plsc.store_scatter(uu, [iota16 * 0 + r, HW + (iota16 & 3)],
                                   pd, mask=iota16 < 4)

        pltpu.sync_copy(dst_hbm.at[pl.ds(start * C, C)], idxd[0])
        pltpu.sync_copy(src_hbm.at[pl.ds(start * C, C)], idxs[0])
        pltpu.sync_copy(dst_hbm.at[pl.ds(start * C, C)], idxd[2])
        pltpu.sync_copy(dst_hbm.at[pl.ds(start * C, C)], idxd[3])
        issue_gathers(0, 0, start * C)
        pltpu.async_copy(dst_hbm.at[pl.ds((start + 1) * C, C)], idxd[1], semi)
        pltpu.async_copy(src_hbm.at[pl.ds((start + 1) * C, C)], idxs[1], semi)
        pltpu.async_copy(uv[0], acc_sh.at[idxd[2]], semsc[0], add=True)
        pltpu.async_copy(uv[1], acc_sh.at[idxd[3]], semsc[1], add=True)

        @pl.loop(0, CPT // 4)
        def _quad(pi):
            for b in range(4):
                ci = 4 * pi + b
                s2 = b % 2
                n2 = (b + 1) % 2
                s4 = b
                n4 = (b + 1) % 4
                p4 = (b + 2) % 4
                base = (start + ci) * C
                nbase = base + C
                pbase = base + 2 * C

                pltpu.make_async_copy(uv[s2], acc_sh.at[idxd[p4]],
                                      semsc[s2]).wait()

                pltpu.make_async_copy(dst_hbm.at[pl.ds(nbase, C)], idxd[n4],
                                      semi).wait()
                pltpu.make_async_copy(src_hbm.at[pl.ds(nbase, C)], idxs[n4],
                                      semi).wait()
                issue_gathers(n2, n4, nbase)

                pltpu.async_copy(dst_hbm.at[pl.ds(pbase, C)], idxd[p4], semi)
                pltpu.async_copy(src_hbm.at[pl.ds(pbase, C)], idxs[p4], semi)

                wait_gathers(s2, s4, base)
                compute(s2)
                pltpu.async_copy(uv[s2], acc_sh.at[idxd[s4]], semsc[s2],
                                 add=True)

        pltpu.make_async_copy(uv[0], acc_sh.at[idxd[2]], semsc[0]).wait()
        pltpu.make_async_copy(uv[1], acc_sh.at[idxd[3]], semsc[1]).wait()
        wait_gathers(0, 0, (start + CPT) * C)
        pltpu.make_async_copy(dst_hbm.at[pl.ds((start + CPT + 1) * C, C)],
                              idxd[1], semi).wait()
        pltpu.make_async_copy(src_hbm.at[pl.ds((start + CPT + 1) * C, C)],
                              idxs[1], semi).wait()

        plsc.subcore_barrier()
        pltpu.sync_copy(acc_sh.at[pl.ds(base_row, ROWS_PER_TILE)],
                        out_hbm.at[qi, cid, pl.ds(base_row, ROWS_PER_TILE)])

    run_pass(0, q0, kv0, w50)
    run_pass(1, q1, kv1, w51)


def _sc_edges(dst, src, qs, kvs, w5s):
    mesh = plsc.VectorSubcoreMesh(core_axis_name="c", subcore_axis_name="s",
                                  num_cores=NC, num_subcores=NS)
    f = pl.kernel(
        _sc_body,
        out_type=jax.ShapeDtypeStruct((NP, NC, NT, CW), jnp.float32),
        mesh=mesh,
        compiler_params=pltpu.CompilerParams(use_tc_tiling_on_sc=False,
                                             needs_layout_passes=False),
        scratch_types=[
            pltpu.VMEM((C,), jnp.int32),
            pltpu.VMEM((C,), jnp.int32),
            pltpu.VMEM((C,), jnp.int32),
            pltpu.VMEM((C,), jnp.int32),
            pltpu.VMEM((C,), jnp.int32),
            pltpu.VMEM((C,), jnp.int32),
            pltpu.VMEM((C,), jnp.int32),
            pltpu.VMEM((C,), jnp.int32),
            pltpu.VMEM((C, HW), jnp.float32),
            pltpu.VMEM((C, HW), jnp.float32),
            pltpu.VMEM((C, 2 * HW), jnp.float32),
            pltpu.VMEM((C, 2 * HW), jnp.float32),
            pltpu.VMEM((C, HW), jnp.float32),
            pltpu.VMEM((C, HW), jnp.float32),
            pltpu.VMEM((C, CW), jnp.float32),
            pltpu.VMEM((C, CW), jnp.float32),
            pltpu.VMEM_SHARED((NT, CW), jnp.float32),
            pltpu.SemaphoreType.DMA,
            pltpu.SemaphoreType.DMA,
            pltpu.SemaphoreType.DMA,
            pltpu.SemaphoreType.DMA,
            pltpu.SemaphoreType.DMA,
        ],
    )
    return f(dst, src, qs[0], kvs[0], w5s[0], qs[1], kvs[1], w5s[1])


def _combine_body(w1_ref, p0_ref, p1_ref, out_ref):
    ri = lax.broadcasted_iota(jnp.int32, (4, HW), 0)
    ci = lax.broadcasted_iota(jnp.int32, (4, HW), 1)
    expand = (ci // HD == ri).astype(jnp.float32)
    outs = []
    for p_ref in (p0_ref, p1_ref):
        p = p_ref[0, 0] + p_ref[0, 1]
        agg = p[:, :HW]
        den = p[:, HW:HW + 4]
        recip = jnp.where(den != 0.0, 1.0 / den, 0.0)
        outs.append(agg * jnp.dot(recip, expand,
                                  preferred_element_type=jnp.float32))
    out_ref[...] = w1_ref[...] + jnp.concatenate(outs, axis=1)


def _combine(w1f, parts):
    grid = (N // NBLK,)

    def pspec(qi):
        return pl.BlockSpec((1, NC, NBLK, CW), lambda i, qi=qi: (qi, 0, i, 0))

    return pl.pallas_call(
        _combine_body,
        grid=grid,
        in_specs=[pl.BlockSpec((NBLK, D), lambda i: (i, 0)),
                  pspec(0), pspec(1)],
        out_specs=pl.BlockSpec((NBLK, D), lambda i: (i, 0)),
        out_shape=jax.ShapeDtypeStruct((N, D), jnp.float32),
    )(w1f, parts, parts)


def kernel(x, edge_index, edge_attr, W1, b1, W2, b2, W3, b3, W4, b4, W5, b5):
    b1r = b1.reshape(1, D)
    b2r = b2.reshape(1, D)
    b3r = b3.reshape(1, D)
    b4r = b4.reshape(1, D)
    b5r = b5.reshape(1, D)
    dst = jnp.concatenate(
        [edge_index[0],
         TRASH + (jnp.arange(EPAD - E, dtype=jnp.int32) % (NT - N))])
    src = jnp.concatenate(
        [edge_index[1], jnp.zeros((EPAD - E,), dtype=jnp.int32)])

    w1f, *nodes = _proj_nodes(x, W1, b1r, W2, b2r, W3, b3r, W4, b4r)
    zq = jnp.zeros((NT - N, HW), jnp.float32)
    qs = [jnp.concatenate([q, zq]) for q in nodes[:NP]]
    kvs = nodes[NP:]
    w5s = _proj_edges(edge_attr, W5, b5r)

    parts = _sc_edges(dst, src, qs, kvs, w5s)

    return _combine(w1f, parts)
